# baseline (speedup 1.0000x reference)
import functools

import jax
import jax.numpy as jnp
from jax import lax
from jax.experimental import pallas as pl
from jax.experimental.pallas import tpu as pltpu
from jax.experimental.pallas import tpu_sc as plsc

BATCH = 4096
EMB_DIM = 64
IMG_FLAT = 3 * 64 * 64
OUT_DIM = IMG_FLAT + EMB_DIM

_NC = 2
_NS = 16
_NW = _NC * _NS
_B_PER_W = BATCH // _NW


def _sc_gather(idx, table):
    mesh = plsc.VectorSubcoreMesh(core_axis_name="c", subcore_axis_name="s")

    @functools.partial(
        pl.kernel,
        mesh=mesh,
        out_type=jax.ShapeDtypeStruct((BATCH, EMB_DIM), jnp.float32),
        scratch_types=[
            pltpu.VMEM((_B_PER_W,), jnp.int32),
            pltpu.VMEM((_B_PER_W, EMB_DIM), jnp.float32),
            pltpu.SemaphoreType.DMA,
        ],
        compiler_params=pltpu.CompilerParams(use_tc_tiling_on_sc=False),
    )
    def gather_kernel(idx_hbm, table_hbm, out_hbm, idx_v, rows_v, sem):
        wid = lax.axis_index("s") * _NC + lax.axis_index("c")
        base = wid * _B_PER_W
        pltpu.sync_copy(idx_hbm.at[pl.ds(base, _B_PER_W)], idx_v)
        pltpu.async_copy(table_hbm.at[idx_v], rows_v, sem).wait()
        pltpu.sync_copy(rows_v, out_hbm.at[pl.ds(base, _B_PER_W)])

    return gather_kernel(idx, table)


def _concat_body(img_ref, memb_ref, out_ref):
    out_ref[:, :IMG_FLAT] = img_ref[...]
    out_ref[:, IMG_FLAT:] = memb_ref[...]


def _tc_concat(img, memb, block_rows):
    return pl.pallas_call(
        _concat_body,
        grid=(BATCH // block_rows,),
        in_specs=[
            pl.BlockSpec((block_rows, IMG_FLAT), lambda i: (i, 0)),
            pl.BlockSpec((block_rows, EMB_DIM), lambda i: (i, 0)),
        ],
        out_specs=pl.BlockSpec((block_rows, OUT_DIM), lambda i: (i, 0)),
        out_shape=jax.ShapeDtypeStruct((BATCH, OUT_DIM), jnp.float32),
    )(img, memb)


def kernel(image, mission, emb):
    img = image.astype(jnp.float32).reshape(image.shape[0], -1)
    idx = mission.astype(jnp.int32)
    memb = _sc_gather(idx, emb)
    return _tc_concat(img, memb, block_rows=128)
